# baseline (speedup 1.0000x reference)
import functools
import math

import jax
import jax.numpy as jnp
from jax import lax
from jax.experimental import pallas as pl
from jax.experimental.pallas import tpu as pltpu

EPS = 1e-5
N_HEADS = 16
FOCUS_K = 64


def _ln(x, g, b):
    mu = jnp.mean(x, axis=-1, keepdims=True)
    var = jnp.mean((x - mu) ** 2, axis=-1, keepdims=True)
    return (x - mu) * lax.rsqrt(var + EPS) * g + b


def _summary_body(h_ref, g_ref, b_ref, out_ref, *, inv_t):
    t = pl.program_id(1)
    x = h_ref[0]
    xn = _ln(x, g_ref[...], b_ref[...])

    @pl.when(t == 0)
    def _():
        out_ref[...] = jnp.zeros_like(out_ref)

    out_ref[0] += jnp.sum(xn, axis=0, keepdims=True) * inv_t


def _select_body(hs_ref, wf_ref, bf_ref, mem_ref, act_ref, aw_ref, out_ref,
                 fq_ref):
    i = pl.program_id(0)

    @pl.when(i == 0)
    def _():
        fq_ref[...] = lax.dot_general(
            hs_ref[...], wf_ref[...], (((1,), (1,)), ((), ())),
            preferred_element_type=jnp.float32) + bf_ref[...]

    rel = lax.dot_general(fq_ref[...], mem_ref[...], (((1,), (1,)), ((), ())),
                          preferred_element_type=jnp.float32)
    out_ref[...] = rel + aw_ref[0, 0] * act_ref[...]


def _topk_body(sel_ref, idx_ref, scratch_ref, *, b, n, k):
    scratch_ref[...] = sel_ref[...]
    iota = lax.broadcasted_iota(jnp.int32, (b, n), 1)
    kcol = lax.broadcasted_iota(jnp.int32, (b, k), 1)

    def step(j, acc):
        vals = scratch_ref[...]
        m = jnp.max(vals, axis=1, keepdims=True)
        idx = jnp.min(jnp.where(vals >= m, iota, n), axis=1, keepdims=True)
        scratch_ref[...] = jnp.where(iota == idx, -jnp.inf, vals)
        return jnp.where(kcol == j, idx, acc)

    idx_ref[...] = lax.fori_loop(0, k, step, jnp.zeros((b, k), jnp.int32))


def _gather_body(idx_ref, mem_ref, out_ref):
    del idx_ref
    out_ref[...] = mem_ref[...]


def _kv_body(tm_ref, wk_ref, bk_ref, wv_ref, bv_ref, k_ref, v_ref):
    tm = tm_ref[...]
    k_ref[...] = lax.dot_general(tm, wk_ref[...], (((1,), (1,)), ((), ())),
                                 preferred_element_type=jnp.float32) + bk_ref[...]
    v_ref[...] = lax.dot_general(tm, wv_ref[...], (((1,), (1,)), ((), ())),
                                 preferred_element_type=jnp.float32) + bv_ref[...]


def _attn_body(h_ref, g_ref, b_ref, wq_ref, bq_ref, k_ref, v_ref, wo_ref,
               bo_ref, gate_ref, out_ref, asum_ref, *, heads, dh, k, t_total):
    t = pl.program_id(1)
    x = h_ref[0]
    xn = _ln(x, g_ref[...], b_ref[...])
    q = lax.dot_general(xn, wq_ref[...], (((1,), (1,)), ((), ())),
                        preferred_element_type=jnp.float32) + bq_ref[...]
    kk = k_ref[0]
    vv = v_ref[0]
    scale = 1.0 / math.sqrt(dh)
    outs = []
    asum = jnp.zeros((1, k), jnp.float32)
    for hh in range(heads):
        qh = q[:, hh * dh:(hh + 1) * dh]
        kh = kk[:, hh * dh:(hh + 1) * dh]
        vh = vv[:, hh * dh:(hh + 1) * dh]
        s = lax.dot_general(qh, kh, (((1,), (1,)), ((), ())),
                            preferred_element_type=jnp.float32) * scale
        s = s - jnp.max(s, axis=1, keepdims=True)
        e = jnp.exp(s)
        p = e / jnp.sum(e, axis=1, keepdims=True)
        outs.append(lax.dot_general(p, vh, (((1,), (0,)), ((), ())),
                                    preferred_element_type=jnp.float32))
        asum = asum + jnp.sum(p, axis=0, keepdims=True)
    att = jnp.concatenate(outs, axis=1)
    o = lax.dot_general(att, wo_ref[...], (((1,), (1,)), ((), ())),
                        preferred_element_type=jnp.float32) + bo_ref[...]
    gate = 1.0 / (1.0 + jnp.exp(-gate_ref[0, 0]))
    out_ref[0] = x + gate * o

    @pl.when(t == 0)
    def _():
        asum_ref[...] = jnp.zeros_like(asum_ref)

    asum_ref[0] += asum * (1.0 / (heads * t_total))


def _scatter_body(idx_ref, val_ref, out_ref, *, k, n):
    idx = idx_ref[0]
    vals = val_ref[0]
    iota = lax.broadcasted_iota(jnp.int32, (k, n), 1)
    onehot = (iota == idx.reshape(k, 1)).astype(jnp.float32)
    out_ref[0] = lax.dot_general(vals, onehot, (((1,), (0,)), ((), ())),
                                 preferred_element_type=jnp.float32)


def kernel(h, memory, activations, Wq, bq, Wk, bk, Wv, bv, Wo, bo, ln_g, ln_b,
           Wf, bf, activation_weight, gate_logit):
    B, T, d = h.shape
    N = memory.shape[0]
    K = min(FOCUS_K, N)
    H = N_HEADS
    Dh = d // H

    g2 = ln_g.reshape(1, d)
    b2 = ln_b.reshape(1, d)
    bq2 = bq.reshape(1, d)
    bf2 = bf.reshape(1, d)
    bk2 = bk.reshape(1, d)
    bv2 = bv.reshape(1, d)
    bo2 = bo.reshape(1, d)
    aw2 = activation_weight.reshape(1, 1)
    gl2 = gate_logit.reshape(1, 1)

    BTS = 512
    h_summary = pl.pallas_call(
        functools.partial(_summary_body, inv_t=1.0 / T),
        grid=(B, T // BTS),
        in_specs=[
            pl.BlockSpec((1, BTS, d), lambda bb, tt: (bb, tt, 0)),
            pl.BlockSpec((1, d), lambda bb, tt: (0, 0)),
            pl.BlockSpec((1, d), lambda bb, tt: (0, 0)),
        ],
        out_specs=pl.BlockSpec((1, 1, d), lambda bb, tt: (bb, 0, 0)),
        out_shape=jax.ShapeDtypeStruct((B, 1, d), jnp.float32),
    )(h, g2, b2)
    h_summary = h_summary.reshape(B, d)

    BN = 2048
    selection = pl.pallas_call(
        _select_body,
        grid=(N // BN,),
        in_specs=[
            pl.BlockSpec((B, d), lambda i: (0, 0)),
            pl.BlockSpec((d, d), lambda i: (0, 0)),
            pl.BlockSpec((1, d), lambda i: (0, 0)),
            pl.BlockSpec((BN, d), lambda i: (i, 0)),
            pl.BlockSpec((B, BN), lambda i: (0, i)),
            pl.BlockSpec((1, 1), lambda i: (0, 0), memory_space=pltpu.SMEM),
        ],
        out_specs=pl.BlockSpec((B, BN), lambda i: (0, i)),
        out_shape=jax.ShapeDtypeStruct((B, N), jnp.float32),
        scratch_shapes=[pltpu.VMEM((B, d), jnp.float32)],
    )(h_summary, Wf, bf2, memory, activations, aw2)

    topk_idx = pl.pallas_call(
        functools.partial(_topk_body, b=B, n=N, k=K),
        in_specs=[pl.BlockSpec((B, N), lambda: (0, 0))],
        out_specs=pl.BlockSpec((B, K), lambda: (0, 0)),
        out_shape=jax.ShapeDtypeStruct((B, K), jnp.int32),
        scratch_shapes=[pltpu.VMEM((B, N), jnp.float32)],
    )(selection)

    idx_flat = topk_idx.reshape(B * K)
    mem3 = memory.reshape(N, 1, d)
    topk_mem = pl.pallas_call(
        _gather_body,
        grid_spec=pltpu.PrefetchScalarGridSpec(
            num_scalar_prefetch=1,
            grid=(B * K,),
            in_specs=[pl.BlockSpec((1, 1, d), lambda i, idx: (idx[i], 0, 0))],
            out_specs=pl.BlockSpec((1, 1, d), lambda i, idx: (i, 0, 0)),
        ),
        out_shape=jax.ShapeDtypeStruct((B * K, 1, d), jnp.float32),
    )(idx_flat, mem3)
    topk_mem = topk_mem.reshape(B * K, d)

    kmat, vmat = pl.pallas_call(
        _kv_body,
        in_specs=[
            pl.BlockSpec((B * K, d), lambda: (0, 0)),
            pl.BlockSpec((d, d), lambda: (0, 0)),
            pl.BlockSpec((1, d), lambda: (0, 0)),
            pl.BlockSpec((d, d), lambda: (0, 0)),
            pl.BlockSpec((1, d), lambda: (0, 0)),
        ],
        out_specs=[
            pl.BlockSpec((B * K, d), lambda: (0, 0)),
            pl.BlockSpec((B * K, d), lambda: (0, 0)),
        ],
        out_shape=[
            jax.ShapeDtypeStruct((B * K, d), jnp.float32),
            jax.ShapeDtypeStruct((B * K, d), jnp.float32),
        ],
    )(topk_mem, Wk, bk2, Wv, bv2)
    kmat = kmat.reshape(B, K, d)
    vmat = vmat.reshape(B, K, d)

    BT = 256
    h_updated, attn_mean = pl.pallas_call(
        functools.partial(_attn_body, heads=H, dh=Dh, k=K, t_total=T),
        grid=(B, T // BT),
        in_specs=[
            pl.BlockSpec((1, BT, d), lambda bb, tt: (bb, tt, 0)),
            pl.BlockSpec((1, d), lambda bb, tt: (0, 0)),
            pl.BlockSpec((1, d), lambda bb, tt: (0, 0)),
            pl.BlockSpec((d, d), lambda bb, tt: (0, 0)),
            pl.BlockSpec((1, d), lambda bb, tt: (0, 0)),
            pl.BlockSpec((1, K, d), lambda bb, tt: (bb, 0, 0)),
            pl.BlockSpec((1, K, d), lambda bb, tt: (bb, 0, 0)),
            pl.BlockSpec((d, d), lambda bb, tt: (0, 0)),
            pl.BlockSpec((1, d), lambda bb, tt: (0, 0)),
            pl.BlockSpec((1, 1), lambda bb, tt: (0, 0), memory_space=pltpu.SMEM),
        ],
        out_specs=[
            pl.BlockSpec((1, BT, d), lambda bb, tt: (bb, tt, 0)),
            pl.BlockSpec((1, 1, K), lambda bb, tt: (bb, 0, 0)),
        ],
        out_shape=[
            jax.ShapeDtypeStruct((B, T, d), jnp.float32),
            jax.ShapeDtypeStruct((B, 1, K), jnp.float32),
        ],
    )(h, g2, b2, Wq, bq2, kmat, vmat, Wo, bo2, gl2)

    idx3 = topk_idx.reshape(B, 1, K)
    val3 = attn_mean
    full_attn = pl.pallas_call(
        functools.partial(_scatter_body, k=K, n=N),
        grid=(B,),
        in_specs=[
            pl.BlockSpec((1, 1, K), lambda bb: (bb, 0, 0)),
            pl.BlockSpec((1, 1, K), lambda bb: (bb, 0, 0)),
        ],
        out_specs=pl.BlockSpec((1, 1, N), lambda bb: (bb, 0, 0)),
        out_shape=jax.ShapeDtypeStruct((B, 1, N), jnp.float32),
    )(idx3, val3)
    full_attn = full_attn.reshape(B, N)

    return h_updated, full_attn


# speedup vs baseline: 1.6379x; 1.6379x over previous
import functools
import math

import jax
import jax.numpy as jnp
from jax import lax
from jax.experimental import pallas as pl
from jax.experimental.pallas import tpu as pltpu
from jax.experimental.pallas import tpu_sc as plsc

EPS = 1e-5
N_HEADS = 16
FOCUS_K = 64


def _ln(x, g, b):
    mu = jnp.mean(x, axis=-1, keepdims=True)
    var = jnp.mean((x - mu) ** 2, axis=-1, keepdims=True)
    return (x - mu) * lax.rsqrt(var + EPS) * g + b


def _summary_body(h_ref, g_ref, b_ref, out_ref, *, inv_t):
    t = pl.program_id(1)
    x = h_ref[0]
    xn = _ln(x, g_ref[...], b_ref[...])

    @pl.when(t == 0)
    def _():
        out_ref[...] = jnp.zeros_like(out_ref)

    out_ref[0] += jnp.sum(xn, axis=0, keepdims=True) * inv_t


def _select_topk_body(hs_ref, wf_ref, bf_ref, mem_ref, act_ref, aw_ref,
                      idx_ref, fq_ref, sel_ref, *, b, n, k, bn, gn):
    i = pl.program_id(0)

    @pl.when(i == 0)
    def _():
        fq_ref[...] = lax.dot_general(
            hs_ref[...], wf_ref[...], (((1,), (1,)), ((), ())),
            preferred_element_type=jnp.float32) + bf_ref[...]

    rel = lax.dot_general(fq_ref[...], mem_ref[...], (((1,), (1,)), ((), ())),
                          preferred_element_type=jnp.float32)
    off = pl.multiple_of(i * bn, bn)
    sel_ref[:, pl.ds(off, bn)] = rel + aw_ref[0, 0] * act_ref[...]

    @pl.when(i == gn - 1)
    def _():
        iota = lax.broadcasted_iota(jnp.int32, (b, n), 1)
        kcol = lax.broadcasted_iota(jnp.int32, (b, k), 1)

        def step(j, acc):
            vals = sel_ref[...]
            m = jnp.max(vals, axis=1, keepdims=True)
            idx = jnp.min(jnp.where(vals >= m, iota, n), axis=1, keepdims=True)
            sel_ref[...] = jnp.where(iota == idx, -jnp.inf, vals)
            return jnp.where(kcol == j, idx, acc)

        idx_ref[...] = lax.fori_loop(0, k, step, jnp.zeros((b, k), jnp.int32))


def _kv_body(tm_ref, wk_ref, bk_ref, wv_ref, bv_ref, k_ref, v_ref):
    tm = tm_ref[...]
    k_ref[...] = lax.dot_general(tm, wk_ref[...], (((1,), (1,)), ((), ())),
                                 preferred_element_type=jnp.float32) + bk_ref[...]
    v_ref[...] = lax.dot_general(tm, wv_ref[...], (((1,), (1,)), ((), ())),
                                 preferred_element_type=jnp.float32) + bv_ref[...]


def _attn_body(h_ref, g_ref, b_ref, wq_ref, bq_ref, k_ref, v_ref, wo_ref,
               bo_ref, gate_ref, out_ref, asum_ref, *, heads, dh, k, t_total):
    t = pl.program_id(1)
    x = h_ref[0]
    xn = _ln(x, g_ref[...], b_ref[...])
    q = lax.dot_general(xn, wq_ref[...], (((1,), (1,)), ((), ())),
                        preferred_element_type=jnp.float32) + bq_ref[...]
    kk = k_ref[0]
    vv = v_ref[0]
    scale = 1.0 / math.sqrt(dh)
    outs = []
    asum = jnp.zeros((1, k), jnp.float32)
    for hh in range(heads):
        qh = q[:, hh * dh:(hh + 1) * dh]
        kh = kk[:, hh * dh:(hh + 1) * dh]
        vh = vv[:, hh * dh:(hh + 1) * dh]
        s = lax.dot_general(qh, kh, (((1,), (1,)), ((), ())),
                            preferred_element_type=jnp.float32) * scale
        s = s - jnp.max(s, axis=1, keepdims=True)
        e = jnp.exp(s)
        p = e / jnp.sum(e, axis=1, keepdims=True)
        outs.append(lax.dot_general(p, vh, (((1,), (0,)), ((), ())),
                                    preferred_element_type=jnp.float32))
        asum = asum + jnp.sum(p, axis=0, keepdims=True)
    att = jnp.concatenate(outs, axis=1)
    o = lax.dot_general(att, wo_ref[...], (((1,), (1,)), ((), ())),
                        preferred_element_type=jnp.float32) + bo_ref[...]
    gate = 1.0 / (1.0 + jnp.exp(-gate_ref[0, 0]))
    out_ref[0] = x + gate * o

    @pl.when(t == 0)
    def _():
        asum_ref[...] = jnp.zeros_like(asum_ref)

    asum_ref[0] += asum * (1.0 / (heads * t_total))


def _scatter_body(idx_ref, val_ref, out_ref, *, k, n):
    idx = idx_ref[0]
    vals = val_ref[0]
    iota = lax.broadcasted_iota(jnp.int32, (k, n), 1)
    onehot = (iota == idx.reshape(k, 1)).astype(jnp.float32)
    out_ref[0] = lax.dot_general(vals, onehot, (((1,), (0,)), ((), ())),
                                 preferred_element_type=jnp.float32)


def _sc_gather(memory, idx_flat, rows, d):
    info = plsc.get_sparse_core_info()
    nw = info.num_cores * info.num_subcores
    b_per_w = rows // nw
    mesh = plsc.VectorSubcoreMesh(core_axis_name="c", subcore_axis_name="s")

    @functools.partial(
        pl.kernel, mesh=mesh,
        out_type=jax.ShapeDtypeStruct((rows, d), jnp.float32),
        scratch_types=[
            pltpu.VMEM((b_per_w,), jnp.int32),
            pltpu.VMEM((b_per_w, d), jnp.float32),
            pltpu.SemaphoreType.DMA,
        ],
    )
    def gk(idx_hbm, mem_hbm, out_hbm, idx_v, rows_v, sem):
        wid = lax.axis_index("s") * info.num_cores + lax.axis_index("c")
        base = wid * b_per_w
        pltpu.sync_copy(idx_hbm.at[pl.ds(base, b_per_w)], idx_v)
        pltpu.async_copy(mem_hbm.at[idx_v], rows_v, sem).wait()
        pltpu.sync_copy(rows_v, out_hbm.at[pl.ds(base, b_per_w)])

    return gk(idx_flat, memory)


def kernel(h, memory, activations, Wq, bq, Wk, bk, Wv, bv, Wo, bo, ln_g, ln_b,
           Wf, bf, activation_weight, gate_logit):
    B, T, d = h.shape
    N = memory.shape[0]
    K = min(FOCUS_K, N)
    H = N_HEADS
    Dh = d // H

    g2 = ln_g.reshape(1, d)
    b2 = ln_b.reshape(1, d)
    bq2 = bq.reshape(1, d)
    bf2 = bf.reshape(1, d)
    bk2 = bk.reshape(1, d)
    bv2 = bv.reshape(1, d)
    bo2 = bo.reshape(1, d)
    aw2 = activation_weight.reshape(1, 1)
    gl2 = gate_logit.reshape(1, 1)

    BTS = 512
    h_summary = pl.pallas_call(
        functools.partial(_summary_body, inv_t=1.0 / T),
        grid=(B, T // BTS),
        in_specs=[
            pl.BlockSpec((1, BTS, d), lambda bb, tt: (bb, tt, 0)),
            pl.BlockSpec((1, d), lambda bb, tt: (0, 0)),
            pl.BlockSpec((1, d), lambda bb, tt: (0, 0)),
        ],
        out_specs=pl.BlockSpec((1, 1, d), lambda bb, tt: (bb, 0, 0)),
        out_shape=jax.ShapeDtypeStruct((B, 1, d), jnp.float32),
    )(h, g2, b2)
    h_summary = h_summary.reshape(B, d)

    BN = 2048
    GN = N // BN
    topk_idx = pl.pallas_call(
        functools.partial(_select_topk_body, b=B, n=N, k=K, bn=BN, gn=GN),
        grid=(GN,),
        in_specs=[
            pl.BlockSpec((B, d), lambda i: (0, 0)),
            pl.BlockSpec((d, d), lambda i: (0, 0)),
            pl.BlockSpec((1, d), lambda i: (0, 0)),
            pl.BlockSpec((BN, d), lambda i: (i, 0)),
            pl.BlockSpec((B, BN), lambda i: (0, i)),
            pl.BlockSpec((1, 1), lambda i: (0, 0), memory_space=pltpu.SMEM),
        ],
        out_specs=pl.BlockSpec((B, K), lambda i: (0, 0)),
        out_shape=jax.ShapeDtypeStruct((B, K), jnp.int32),
        scratch_shapes=[
            pltpu.VMEM((B, d), jnp.float32),
            pltpu.VMEM((B, N), jnp.float32),
        ],
    )(h_summary, Wf, bf2, memory, activations, aw2)

    idx_flat = topk_idx.reshape(B * K)
    topk_mem = _sc_gather(memory, idx_flat, B * K, d)

    kmat, vmat = pl.pallas_call(
        _kv_body,
        in_specs=[
            pl.BlockSpec((B * K, d), lambda: (0, 0)),
            pl.BlockSpec((d, d), lambda: (0, 0)),
            pl.BlockSpec((1, d), lambda: (0, 0)),
            pl.BlockSpec((d, d), lambda: (0, 0)),
            pl.BlockSpec((1, d), lambda: (0, 0)),
        ],
        out_specs=[
            pl.BlockSpec((B * K, d), lambda: (0, 0)),
            pl.BlockSpec((B * K, d), lambda: (0, 0)),
        ],
        out_shape=[
            jax.ShapeDtypeStruct((B * K, d), jnp.float32),
            jax.ShapeDtypeStruct((B * K, d), jnp.float32),
        ],
    )(topk_mem, Wk, bk2, Wv, bv2)
    kmat = kmat.reshape(B, K, d)
    vmat = vmat.reshape(B, K, d)

    BT = 512
    h_updated, attn_mean = pl.pallas_call(
        functools.partial(_attn_body, heads=H, dh=Dh, k=K, t_total=T),
        grid=(B, T // BT),
        in_specs=[
            pl.BlockSpec((1, BT, d), lambda bb, tt: (bb, tt, 0)),
            pl.BlockSpec((1, d), lambda bb, tt: (0, 0)),
            pl.BlockSpec((1, d), lambda bb, tt: (0, 0)),
            pl.BlockSpec((d, d), lambda bb, tt: (0, 0)),
            pl.BlockSpec((1, d), lambda bb, tt: (0, 0)),
            pl.BlockSpec((1, K, d), lambda bb, tt: (bb, 0, 0)),
            pl.BlockSpec((1, K, d), lambda bb, tt: (bb, 0, 0)),
            pl.BlockSpec((d, d), lambda bb, tt: (0, 0)),
            pl.BlockSpec((1, d), lambda bb, tt: (0, 0)),
            pl.BlockSpec((1, 1), lambda bb, tt: (0, 0), memory_space=pltpu.SMEM),
        ],
        out_specs=[
            pl.BlockSpec((1, BT, d), lambda bb, tt: (bb, tt, 0)),
            pl.BlockSpec((1, 1, K), lambda bb, tt: (bb, 0, 0)),
        ],
        out_shape=[
            jax.ShapeDtypeStruct((B, T, d), jnp.float32),
            jax.ShapeDtypeStruct((B, 1, K), jnp.float32),
        ],
    )(h, g2, b2, Wq, bq2, kmat, vmat, Wo, bo2, gl2)

    idx3 = topk_idx.reshape(B, 1, K)
    full_attn = pl.pallas_call(
        functools.partial(_scatter_body, k=K, n=N),
        grid=(B,),
        in_specs=[
            pl.BlockSpec((1, 1, K), lambda bb: (bb, 0, 0)),
            pl.BlockSpec((1, 1, K), lambda bb: (bb, 0, 0)),
        ],
        out_specs=pl.BlockSpec((1, 1, N), lambda bb: (bb, 0, 0)),
        out_shape=jax.ShapeDtypeStruct((B, 1, N), jnp.float32),
    )(idx3, attn_mean)
    full_attn = full_attn.reshape(B, N)

    return h_updated, full_attn


# speedup vs baseline: 1.7313x; 1.0570x over previous
import functools
import math

import jax
import jax.numpy as jnp
from jax import lax
from jax.experimental import pallas as pl
from jax.experimental.pallas import tpu as pltpu
from jax.experimental.pallas import tpu_sc as plsc

EPS = 1e-5
N_HEADS = 16
FOCUS_K = 64


def _ln(x, g, b):
    mu = jnp.mean(x, axis=-1, keepdims=True)
    var = jnp.mean((x - mu) ** 2, axis=-1, keepdims=True)
    return (x - mu) * lax.rsqrt(var + EPS) * g + b


def _summary_body(h_ref, g_ref, b_ref, out_ref, *, inv_t):
    t = pl.program_id(1)
    x = h_ref[0]
    xn = _ln(x, g_ref[...], b_ref[...])

    @pl.when(t == 0)
    def _():
        out_ref[...] = jnp.zeros_like(out_ref)

    out_ref[0] += jnp.sum(xn, axis=0, keepdims=True) * inv_t


def _select_topk_body(hs_ref, wf_ref, bf_ref, mem_ref, act_ref, aw_ref,
                      idx_ref, fq_ref, sel_ref, *, b, n, k, bn, gn):
    i = pl.program_id(0)

    @pl.when(i == 0)
    def _():
        fq_ref[...] = lax.dot_general(
            hs_ref[...], wf_ref[...], (((1,), (1,)), ((), ())),
            preferred_element_type=jnp.float32) + bf_ref[...]

    rel = lax.dot_general(fq_ref[...], mem_ref[...], (((1,), (1,)), ((), ())),
                          preferred_element_type=jnp.float32)
    off = pl.multiple_of(i * bn, bn)
    sel_ref[:, pl.ds(off, bn)] = rel + aw_ref[0, 0] * act_ref[...]

    @pl.when(i == gn - 1)
    def _():
        iota = lax.broadcasted_iota(jnp.int32, (b, n), 1)
        kcol = lax.broadcasted_iota(jnp.int32, (b, k), 1)

        def step(j, acc):
            vals = sel_ref[...]
            m = jnp.max(vals, axis=1, keepdims=True)
            idx = jnp.min(jnp.where(vals >= m, iota, n), axis=1, keepdims=True)
            sel_ref[...] = jnp.where(iota == idx, -jnp.inf, vals)
            return jnp.where(kcol == j, idx, acc)

        idx_ref[...] = lax.fori_loop(0, k, step, jnp.zeros((b, k), jnp.int32))


def _kv_body(tm_ref, wk_ref, bk_ref, wv_ref, bv_ref, k_ref, v_ref):
    tm = tm_ref[...]
    kf = lax.dot_general(tm, wk_ref[...], (((1,), (1,)), ((), ())),
                         preferred_element_type=jnp.float32) + bk_ref[...]
    vf = lax.dot_general(tm, wv_ref[...], (((1,), (1,)), ((), ())),
                         preferred_element_type=jnp.float32) + bv_ref[...]
    k_ref[...] = kf.astype(jnp.bfloat16)
    v_ref[...] = vf.astype(jnp.bfloat16)


def _attn_body(h_ref, g_ref, b_ref, wq_ref, bq_ref, k_ref, v_ref, wo_ref,
               bo_ref, gate_ref, out_ref, asum_ref, *, heads, dh, k, t_total):
    t = pl.program_id(1)
    x = h_ref[0]
    xn = _ln(x, g_ref[...], b_ref[...])
    q = lax.dot_general(xn.astype(jnp.bfloat16), wq_ref[...],
                        (((1,), (1,)), ((), ())),
                        preferred_element_type=jnp.float32) + bq_ref[...]
    qb = q.astype(jnp.bfloat16)
    kk = k_ref[0]
    vv = v_ref[0]
    scale = 1.0 / math.sqrt(dh)
    outs = []
    asum = jnp.zeros((1, k), jnp.float32)
    for hh in range(heads):
        qh = qb[:, hh * dh:(hh + 1) * dh]
        kh = kk[:, hh * dh:(hh + 1) * dh]
        vh = vv[:, hh * dh:(hh + 1) * dh]
        s = lax.dot_general(qh, kh, (((1,), (1,)), ((), ())),
                            preferred_element_type=jnp.float32) * scale
        s = s - jnp.max(s, axis=1, keepdims=True)
        e = jnp.exp(s)
        p = e / jnp.sum(e, axis=1, keepdims=True)
        outs.append(lax.dot_general(p.astype(jnp.bfloat16), vh,
                                    (((1,), (0,)), ((), ())),
                                    preferred_element_type=jnp.float32))
        asum = asum + jnp.sum(p, axis=0, keepdims=True)
    att = jnp.concatenate(outs, axis=1).astype(jnp.bfloat16)
    o = lax.dot_general(att, wo_ref[...], (((1,), (1,)), ((), ())),
                        preferred_element_type=jnp.float32) + bo_ref[...]
    gate = 1.0 / (1.0 + jnp.exp(-gate_ref[0, 0]))
    out_ref[0] = x + gate * o

    @pl.when(t == 0)
    def _():
        asum_ref[...] = jnp.zeros_like(asum_ref)

    asum_ref[0] += asum * (1.0 / (heads * t_total))


def _scatter_body(idx_ref, val_ref, out_ref, *, k, n):
    idx = idx_ref[0]
    vals = val_ref[0]
    iota = lax.broadcasted_iota(jnp.int32, (k, n), 1)
    onehot = (iota == idx.reshape(k, 1)).astype(jnp.float32)
    out_ref[0] = lax.dot_general(vals, onehot, (((1,), (0,)), ((), ())),
                                 preferred_element_type=jnp.float32)


def _sc_gather(memory, idx_flat, rows, d):
    info = plsc.get_sparse_core_info()
    nw = info.num_cores * info.num_subcores
    b_per_w = rows // nw
    mesh = plsc.VectorSubcoreMesh(core_axis_name="c", subcore_axis_name="s")

    @functools.partial(
        pl.kernel, mesh=mesh,
        out_type=jax.ShapeDtypeStruct((rows, d), jnp.float32),
        scratch_types=[
            pltpu.VMEM((b_per_w,), jnp.int32),
            pltpu.VMEM((b_per_w, d), jnp.float32),
            pltpu.SemaphoreType.DMA,
        ],
    )
    def gk(idx_hbm, mem_hbm, out_hbm, idx_v, rows_v, sem):
        wid = lax.axis_index("s") * info.num_cores + lax.axis_index("c")
        base = wid * b_per_w
        pltpu.sync_copy(idx_hbm.at[pl.ds(base, b_per_w)], idx_v)
        pltpu.async_copy(mem_hbm.at[idx_v], rows_v, sem).wait()
        pltpu.sync_copy(rows_v, out_hbm.at[pl.ds(base, b_per_w)])

    return gk(idx_flat, memory)


def kernel(h, memory, activations, Wq, bq, Wk, bk, Wv, bv, Wo, bo, ln_g, ln_b,
           Wf, bf, activation_weight, gate_logit):
    B, T, d = h.shape
    N = memory.shape[0]
    K = min(FOCUS_K, N)
    H = N_HEADS
    Dh = d // H

    g2 = ln_g.reshape(1, d)
    b2 = ln_b.reshape(1, d)
    bq2 = bq.reshape(1, d)
    bf2 = bf.reshape(1, d)
    bk2 = bk.reshape(1, d)
    bv2 = bv.reshape(1, d)
    bo2 = bo.reshape(1, d)
    aw2 = activation_weight.reshape(1, 1)
    gl2 = gate_logit.reshape(1, 1)

    BTS = 512
    h_summary = pl.pallas_call(
        functools.partial(_summary_body, inv_t=1.0 / T),
        grid=(B, T // BTS),
        in_specs=[
            pl.BlockSpec((1, BTS, d), lambda bb, tt: (bb, tt, 0)),
            pl.BlockSpec((1, d), lambda bb, tt: (0, 0)),
            pl.BlockSpec((1, d), lambda bb, tt: (0, 0)),
        ],
        out_specs=pl.BlockSpec((1, 1, d), lambda bb, tt: (bb, 0, 0)),
        out_shape=jax.ShapeDtypeStruct((B, 1, d), jnp.float32),
    )(h, g2, b2)
    h_summary = h_summary.reshape(B, d)

    BN = 2048
    GN = N // BN
    topk_idx = pl.pallas_call(
        functools.partial(_select_topk_body, b=B, n=N, k=K, bn=BN, gn=GN),
        grid=(GN,),
        in_specs=[
            pl.BlockSpec((B, d), lambda i: (0, 0)),
            pl.BlockSpec((d, d), lambda i: (0, 0)),
            pl.BlockSpec((1, d), lambda i: (0, 0)),
            pl.BlockSpec((BN, d), lambda i: (i, 0)),
            pl.BlockSpec((B, BN), lambda i: (0, i)),
            pl.BlockSpec((1, 1), lambda i: (0, 0), memory_space=pltpu.SMEM),
        ],
        out_specs=pl.BlockSpec((B, K), lambda i: (0, 0)),
        out_shape=jax.ShapeDtypeStruct((B, K), jnp.int32),
        scratch_shapes=[
            pltpu.VMEM((B, d), jnp.float32),
            pltpu.VMEM((B, N), jnp.float32),
        ],
    )(h_summary, Wf, bf2, memory, activations, aw2)

    idx_flat = topk_idx.reshape(B * K)
    topk_mem = _sc_gather(memory, idx_flat, B * K, d)

    kmat, vmat = pl.pallas_call(
        _kv_body,
        in_specs=[
            pl.BlockSpec((B * K, d), lambda: (0, 0)),
            pl.BlockSpec((d, d), lambda: (0, 0)),
            pl.BlockSpec((1, d), lambda: (0, 0)),
            pl.BlockSpec((d, d), lambda: (0, 0)),
            pl.BlockSpec((1, d), lambda: (0, 0)),
        ],
        out_specs=[
            pl.BlockSpec((B * K, d), lambda: (0, 0)),
            pl.BlockSpec((B * K, d), lambda: (0, 0)),
        ],
        out_shape=[
            jax.ShapeDtypeStruct((B * K, d), jnp.bfloat16),
            jax.ShapeDtypeStruct((B * K, d), jnp.bfloat16),
        ],
    )(topk_mem, Wk, bk2, Wv, bv2)
    kmat = kmat.reshape(B, K, d)
    vmat = vmat.reshape(B, K, d)

    BT = 512
    h_updated, attn_mean = pl.pallas_call(
        functools.partial(_attn_body, heads=H, dh=Dh, k=K, t_total=T),
        grid=(B, T // BT),
        in_specs=[
            pl.BlockSpec((1, BT, d), lambda bb, tt: (bb, tt, 0)),
            pl.BlockSpec((1, d), lambda bb, tt: (0, 0)),
            pl.BlockSpec((1, d), lambda bb, tt: (0, 0)),
            pl.BlockSpec((d, d), lambda bb, tt: (0, 0)),
            pl.BlockSpec((1, d), lambda bb, tt: (0, 0)),
            pl.BlockSpec((1, K, d), lambda bb, tt: (bb, 0, 0)),
            pl.BlockSpec((1, K, d), lambda bb, tt: (bb, 0, 0)),
            pl.BlockSpec((d, d), lambda bb, tt: (0, 0)),
            pl.BlockSpec((1, d), lambda bb, tt: (0, 0)),
            pl.BlockSpec((1, 1), lambda bb, tt: (0, 0), memory_space=pltpu.SMEM),
        ],
        out_specs=[
            pl.BlockSpec((1, BT, d), lambda bb, tt: (bb, tt, 0)),
            pl.BlockSpec((1, 1, K), lambda bb, tt: (bb, 0, 0)),
        ],
        out_shape=[
            jax.ShapeDtypeStruct((B, T, d), jnp.float32),
            jax.ShapeDtypeStruct((B, 1, K), jnp.float32),
        ],
    )(h, g2, b2, Wq.astype(jnp.bfloat16), bq2, kmat, vmat,
      Wo.astype(jnp.bfloat16), bo2, gl2)

    idx3 = topk_idx.reshape(B, 1, K)
    full_attn = pl.pallas_call(
        functools.partial(_scatter_body, k=K, n=N),
        grid=(B,),
        in_specs=[
            pl.BlockSpec((1, 1, K), lambda bb: (bb, 0, 0)),
            pl.BlockSpec((1, 1, K), lambda bb: (bb, 0, 0)),
        ],
        out_specs=pl.BlockSpec((1, 1, N), lambda bb: (bb, 0, 0)),
        out_shape=jax.ShapeDtypeStruct((B, 1, N), jnp.float32),
    )(idx3, attn_mean)
    full_attn = full_attn.reshape(B, N)

    return h_updated, full_attn


# speedup vs baseline: 1.7571x; 1.0149x over previous
import functools
import math

import jax
import jax.numpy as jnp
from jax import lax
from jax.experimental import pallas as pl
from jax.experimental.pallas import tpu as pltpu
from jax.experimental.pallas import tpu_sc as plsc

EPS = 1e-5
N_HEADS = 16
FOCUS_K = 64


def _ln(x, g, b):
    mu = jnp.mean(x, axis=-1, keepdims=True)
    var = jnp.mean((x - mu) ** 2, axis=-1, keepdims=True)
    return (x - mu) * lax.rsqrt(var + EPS) * g + b


def _summary_body(h_ref, g_ref, b_ref, out_ref, *, inv_t):
    t = pl.program_id(1)
    x = h_ref[0]
    xn = _ln(x, g_ref[...], b_ref[...])

    @pl.when(t == 0)
    def _():
        out_ref[...] = jnp.zeros_like(out_ref)

    out_ref[0] += jnp.sum(xn, axis=0, keepdims=True) * inv_t


def _select_topk_body(hs_ref, wf_ref, bf_ref, mem_ref, act_ref, aw_ref,
                      idx_ref, fq_ref, sel_ref, *, b, n, k, bn, gn):
    i = pl.program_id(0)

    @pl.when(i == 0)
    def _():
        fq_ref[...] = lax.dot_general(
            hs_ref[...], wf_ref[...], (((1,), (1,)), ((), ())),
            preferred_element_type=jnp.float32) + bf_ref[...]

    rel = lax.dot_general(fq_ref[...], mem_ref[...], (((1,), (1,)), ((), ())),
                          preferred_element_type=jnp.float32)
    off = pl.multiple_of(i * bn, bn)
    sel_ref[:, pl.ds(off, bn)] = rel + aw_ref[0, 0] * act_ref[...]

    @pl.when(i == gn - 1)
    def _():
        iota = lax.broadcasted_iota(jnp.int32, (b, n), 1)
        kcol = lax.broadcasted_iota(jnp.int32, (b, k), 1)

        def step(j, acc):
            vals = sel_ref[...]
            m = jnp.max(vals, axis=1, keepdims=True)
            idx = jnp.min(jnp.where(vals >= m, iota, n), axis=1, keepdims=True)
            sel_ref[...] = jnp.where(iota == idx, -jnp.inf, vals)
            return jnp.where(kcol == j, idx, acc)

        idx_ref[...] = lax.fori_loop(0, k, step, jnp.zeros((b, k), jnp.int32))


def _attn_body(h_ref, g_ref, b_ref, wq_ref, bq_ref, tm_ref, wk_ref, bk_ref,
               wv_ref, bv_ref, wo_ref, bo_ref, gate_ref, idx_ref,
               out_ref, fa_ref, kv_k, kv_v, asum_ref,
               *, heads, dh, k, n, t_total, t_steps):
    b = pl.program_id(0)
    t = pl.program_id(1)

    @pl.when((b == 0) & (t == 0))
    def _():
        tm = tm_ref[...]
        kf = lax.dot_general(tm, wk_ref[...], (((1,), (1,)), ((), ())),
                             preferred_element_type=jnp.float32) + bk_ref[...]
        vf = lax.dot_general(tm, wv_ref[...], (((1,), (1,)), ((), ())),
                             preferred_element_type=jnp.float32) + bv_ref[...]
        kv_k[...] = kf.astype(jnp.bfloat16)
        kv_v[...] = vf.astype(jnp.bfloat16)

    x = h_ref[0]
    xn = _ln(x, g_ref[...], b_ref[...])
    q = lax.dot_general(xn.astype(jnp.bfloat16), wq_ref[...],
                        (((1,), (1,)), ((), ())),
                        preferred_element_type=jnp.float32) + bq_ref[...]
    qb = q.astype(jnp.bfloat16)
    kk = kv_k[pl.ds(pl.multiple_of(b * k, k), k), :]
    vv = kv_v[pl.ds(pl.multiple_of(b * k, k), k), :]
    scale = 1.0 / math.sqrt(dh)
    outs = []
    asum = jnp.zeros((1, k), jnp.float32)
    for hh in range(heads):
        qh = qb[:, hh * dh:(hh + 1) * dh]
        kh = kk[:, hh * dh:(hh + 1) * dh]
        vh = vv[:, hh * dh:(hh + 1) * dh]
        s = lax.dot_general(qh, kh, (((1,), (1,)), ((), ())),
                            preferred_element_type=jnp.float32) * scale
        s = s - jnp.max(s, axis=1, keepdims=True)
        e = jnp.exp(s)
        p = e / jnp.sum(e, axis=1, keepdims=True)
        outs.append(lax.dot_general(p.astype(jnp.bfloat16), vh,
                                    (((1,), (0,)), ((), ())),
                                    preferred_element_type=jnp.float32))
        asum = asum + jnp.sum(p, axis=0, keepdims=True)
    att = jnp.concatenate(outs, axis=1).astype(jnp.bfloat16)
    o = lax.dot_general(att, wo_ref[...], (((1,), (1,)), ((), ())),
                        preferred_element_type=jnp.float32) + bo_ref[...]
    gate = 1.0 / (1.0 + jnp.exp(-gate_ref[0, 0]))
    out_ref[0] = x + gate * o

    @pl.when(t == 0)
    def _():
        asum_ref[...] = jnp.zeros_like(asum_ref)

    asum_ref[...] += asum * (1.0 / (heads * t_total))

    @pl.when(t == t_steps - 1)
    def _():
        idx = idx_ref[0]
        vals = asum_ref[...]
        iota = lax.broadcasted_iota(jnp.int32, (k, n), 1)
        onehot = (iota == idx.reshape(k, 1)).astype(jnp.float32)
        fa_ref[0] = lax.dot_general(vals, onehot, (((1,), (0,)), ((), ())),
                                    preferred_element_type=jnp.float32)


def _sc_gather(memory, idx_flat, rows, d):
    info = plsc.get_sparse_core_info()
    nw = info.num_cores * info.num_subcores
    b_per_w = rows // nw
    mesh = plsc.VectorSubcoreMesh(core_axis_name="c", subcore_axis_name="s")

    @functools.partial(
        pl.kernel, mesh=mesh,
        out_type=jax.ShapeDtypeStruct((rows, d), jnp.float32),
        scratch_types=[
            pltpu.VMEM((b_per_w,), jnp.int32),
            pltpu.VMEM((b_per_w, d), jnp.float32),
            pltpu.SemaphoreType.DMA,
        ],
    )
    def gk(idx_hbm, mem_hbm, out_hbm, idx_v, rows_v, sem):
        wid = lax.axis_index("s") * info.num_cores + lax.axis_index("c")
        base = wid * b_per_w
        pltpu.sync_copy(idx_hbm.at[pl.ds(base, b_per_w)], idx_v)
        pltpu.async_copy(mem_hbm.at[idx_v], rows_v, sem).wait()
        pltpu.sync_copy(rows_v, out_hbm.at[pl.ds(base, b_per_w)])

    return gk(idx_flat, memory)


def kernel(h, memory, activations, Wq, bq, Wk, bk, Wv, bv, Wo, bo, ln_g, ln_b,
           Wf, bf, activation_weight, gate_logit):
    B, T, d = h.shape
    N = memory.shape[0]
    K = min(FOCUS_K, N)
    H = N_HEADS
    Dh = d // H

    g2 = ln_g.reshape(1, d)
    b2 = ln_b.reshape(1, d)
    bq2 = bq.reshape(1, d)
    bf2 = bf.reshape(1, d)
    bk2 = bk.reshape(1, d)
    bv2 = bv.reshape(1, d)
    bo2 = bo.reshape(1, d)
    aw2 = activation_weight.reshape(1, 1)
    gl2 = gate_logit.reshape(1, 1)

    BTS = 512
    h_summary = pl.pallas_call(
        functools.partial(_summary_body, inv_t=1.0 / T),
        grid=(B, T // BTS),
        in_specs=[
            pl.BlockSpec((1, BTS, d), lambda bb, tt: (bb, tt, 0)),
            pl.BlockSpec((1, d), lambda bb, tt: (0, 0)),
            pl.BlockSpec((1, d), lambda bb, tt: (0, 0)),
        ],
        out_specs=pl.BlockSpec((1, 1, d), lambda bb, tt: (bb, 0, 0)),
        out_shape=jax.ShapeDtypeStruct((B, 1, d), jnp.float32),
    )(h, g2, b2)
    h_summary = h_summary.reshape(B, d)

    BN = 2048
    GN = N // BN
    topk_idx = pl.pallas_call(
        functools.partial(_select_topk_body, b=B, n=N, k=K, bn=BN, gn=GN),
        grid=(GN,),
        in_specs=[
            pl.BlockSpec((B, d), lambda i: (0, 0)),
            pl.BlockSpec((d, d), lambda i: (0, 0)),
            pl.BlockSpec((1, d), lambda i: (0, 0)),
            pl.BlockSpec((BN, d), lambda i: (i, 0)),
            pl.BlockSpec((B, BN), lambda i: (0, i)),
            pl.BlockSpec((1, 1), lambda i: (0, 0), memory_space=pltpu.SMEM),
        ],
        out_specs=pl.BlockSpec((B, K), lambda i: (0, 0)),
        out_shape=jax.ShapeDtypeStruct((B, K), jnp.int32),
        scratch_shapes=[
            pltpu.VMEM((B, d), jnp.float32),
            pltpu.VMEM((B, N), jnp.float32),
        ],
    )(h_summary, Wf, bf2, memory, activations, aw2)

    idx_flat = topk_idx.reshape(B * K)
    topk_mem = _sc_gather(memory, idx_flat, B * K, d)

    BT = 512
    TS = T // BT
    idx3 = topk_idx.reshape(B, 1, K)
    h_updated, full_attn = pl.pallas_call(
        functools.partial(_attn_body, heads=H, dh=Dh, k=K, n=N, t_total=T,
                          t_steps=TS),
        grid=(B, TS),
        in_specs=[
            pl.BlockSpec((1, BT, d), lambda bb, tt: (bb, tt, 0)),
            pl.BlockSpec((1, d), lambda bb, tt: (0, 0)),
            pl.BlockSpec((1, d), lambda bb, tt: (0, 0)),
            pl.BlockSpec((d, d), lambda bb, tt: (0, 0)),
            pl.BlockSpec((1, d), lambda bb, tt: (0, 0)),
            pl.BlockSpec((B * K, d), lambda bb, tt: (0, 0)),
            pl.BlockSpec((d, d), lambda bb, tt: (0, 0)),
            pl.BlockSpec((1, d), lambda bb, tt: (0, 0)),
            pl.BlockSpec((d, d), lambda bb, tt: (0, 0)),
            pl.BlockSpec((1, d), lambda bb, tt: (0, 0)),
            pl.BlockSpec((d, d), lambda bb, tt: (0, 0)),
            pl.BlockSpec((1, d), lambda bb, tt: (0, 0)),
            pl.BlockSpec((1, 1), lambda bb, tt: (0, 0), memory_space=pltpu.SMEM),
            pl.BlockSpec((1, 1, K), lambda bb, tt: (bb, 0, 0)),
        ],
        out_specs=[
            pl.BlockSpec((1, BT, d), lambda bb, tt: (bb, tt, 0)),
            pl.BlockSpec((1, 1, N), lambda bb, tt: (bb, 0, 0)),
        ],
        out_shape=[
            jax.ShapeDtypeStruct((B, T, d), jnp.float32),
            jax.ShapeDtypeStruct((B, 1, N), jnp.float32),
        ],
        scratch_shapes=[
            pltpu.VMEM((B * K, d), jnp.bfloat16),
            pltpu.VMEM((B * K, d), jnp.bfloat16),
            pltpu.VMEM((1, K), jnp.float32),
        ],
    )(h, g2, b2, Wq.astype(jnp.bfloat16), bq2, topk_mem, Wk, bk2, Wv, bv2,
      Wo.astype(jnp.bfloat16), bo2, gl2, idx3)
    full_attn = full_attn.reshape(B, N)

    return h_updated, full_attn


# speedup vs baseline: 1.7965x; 1.0224x over previous
import functools
import math

import jax
import jax.numpy as jnp
from jax import lax
from jax.experimental import pallas as pl
from jax.experimental.pallas import tpu as pltpu
from jax.experimental.pallas import tpu_sc as plsc

EPS = 1e-5
N_HEADS = 16
FOCUS_K = 64


def _ln(x, g, b):
    mu = jnp.mean(x, axis=-1, keepdims=True)
    var = jnp.mean((x - mu) ** 2, axis=-1, keepdims=True)
    return (x - mu) * lax.rsqrt(var + EPS) * g + b


def _sumsel_body(h_ref, g_ref, b_ref, wf_ref, bf_ref, act_ref, aw_ref,
                 mem_hbm, idx_ref, sums_ref, mem_v, sem, sel_ref,
                 *, bsz, n, k, t_steps, inv_t):
    bb = pl.program_id(0)
    t = pl.program_id(1)

    @pl.when((bb == 0) & (t == 0))
    def _():
        sums_ref[...] = jnp.zeros_like(sums_ref)
        pltpu.make_async_copy(mem_hbm, mem_v, sem).start()

    x = h_ref[0]
    xn = _ln(x, g_ref[...], b_ref[...])
    part = jnp.sum(xn, axis=0, keepdims=True) * inv_t
    biota = lax.broadcasted_iota(jnp.int32, (bsz, 1), 0)
    sums_ref[...] += jnp.where(biota == bb, part, 0.0)

    @pl.when((bb == bsz - 1) & (t == t_steps - 1))
    def _():
        pltpu.make_async_copy(mem_hbm, mem_v, sem).wait()
        fq = lax.dot_general(
            sums_ref[...], wf_ref[...], (((1,), (1,)), ((), ())),
            preferred_element_type=jnp.float32) + bf_ref[...]
        rel = lax.dot_general(fq, mem_v[...], (((1,), (1,)), ((), ())),
                              preferred_element_type=jnp.float32)
        sel_ref[...] = rel + aw_ref[0, 0] * act_ref[...]

        iota = lax.broadcasted_iota(jnp.int32, (bsz, n), 1)
        kcol = lax.broadcasted_iota(jnp.int32, (bsz, k), 1)

        def step(j, acc):
            vals = sel_ref[...]
            m = jnp.max(vals, axis=1, keepdims=True)
            idx = jnp.min(jnp.where(vals >= m, iota, n), axis=1, keepdims=True)
            sel_ref[...] = jnp.where(iota == idx, -jnp.inf, vals)
            return jnp.where(kcol == j, idx, acc)

        idx_ref[...] = lax.fori_loop(0, k, step,
                                     jnp.zeros((bsz, k), jnp.int32))


def _attn_body(h_ref, g_ref, b_ref, wq_ref, bq_ref, tm_ref, wk_ref, bk_ref,
               wv_ref, bv_ref, wo_ref, bo_ref, gate_ref, idx_ref,
               out_ref, fa_ref, kv_k, kv_v, asum_ref, wq_b, wo_b,
               *, heads, dh, k, n, t_total, t_steps):
    b = pl.program_id(0)
    t = pl.program_id(1)

    @pl.when((b == 0) & (t == 0))
    def _():
        tm = tm_ref[...]
        kf = lax.dot_general(tm, wk_ref[...], (((1,), (1,)), ((), ())),
                             preferred_element_type=jnp.float32) + bk_ref[...]
        vf = lax.dot_general(tm, wv_ref[...], (((1,), (1,)), ((), ())),
                             preferred_element_type=jnp.float32) + bv_ref[...]
        kv_k[...] = kf.astype(jnp.bfloat16)
        kv_v[...] = vf.astype(jnp.bfloat16)
        wq_b[...] = wq_ref[...].astype(jnp.bfloat16)
        wo_b[...] = wo_ref[...].astype(jnp.bfloat16)

    x = h_ref[0]
    xn = _ln(x, g_ref[...], b_ref[...])
    q = lax.dot_general(xn.astype(jnp.bfloat16), wq_b[...],
                        (((1,), (1,)), ((), ())),
                        preferred_element_type=jnp.float32) + bq_ref[...]
    qb = q.astype(jnp.bfloat16)
    kk = kv_k[pl.ds(pl.multiple_of(b * k, k), k), :]
    vv = kv_v[pl.ds(pl.multiple_of(b * k, k), k), :]
    scale = 1.0 / math.sqrt(dh)
    outs = []
    asum = jnp.zeros((1, k), jnp.float32)
    for hh in range(heads):
        qh = qb[:, hh * dh:(hh + 1) * dh]
        kh = kk[:, hh * dh:(hh + 1) * dh]
        vh = vv[:, hh * dh:(hh + 1) * dh]
        s = lax.dot_general(qh, kh, (((1,), (1,)), ((), ())),
                            preferred_element_type=jnp.float32) * scale
        s = s - jnp.max(s, axis=1, keepdims=True)
        e = jnp.exp(s)
        p = e / jnp.sum(e, axis=1, keepdims=True)
        outs.append(lax.dot_general(p.astype(jnp.bfloat16), vh,
                                    (((1,), (0,)), ((), ())),
                                    preferred_element_type=jnp.float32))
        asum = asum + jnp.sum(p, axis=0, keepdims=True)
    att = jnp.concatenate(outs, axis=1).astype(jnp.bfloat16)
    o = lax.dot_general(att, wo_b[...], (((1,), (1,)), ((), ())),
                        preferred_element_type=jnp.float32) + bo_ref[...]
    gate = 1.0 / (1.0 + jnp.exp(-gate_ref[0, 0]))
    out_ref[0] = x + gate * o

    @pl.when(t == 0)
    def _():
        asum_ref[...] = jnp.zeros_like(asum_ref)

    asum_ref[...] += asum * (1.0 / (heads * t_total))

    @pl.when(t == t_steps - 1)
    def _():
        idx = idx_ref[0]
        vals = asum_ref[...]
        iota = lax.broadcasted_iota(jnp.int32, (k, n), 1)
        onehot = (iota == idx.reshape(k, 1)).astype(jnp.float32)
        fa_ref[0] = lax.dot_general(vals, onehot, (((1,), (0,)), ((), ())),
                                    preferred_element_type=jnp.float32)


def _sc_gather(memory, idx_flat, rows, d):
    info = plsc.get_sparse_core_info()
    nw = info.num_cores * info.num_subcores
    b_per_w = rows // nw
    mesh = plsc.VectorSubcoreMesh(core_axis_name="c", subcore_axis_name="s")

    @functools.partial(
        pl.kernel, mesh=mesh,
        out_type=jax.ShapeDtypeStruct((rows, d), jnp.float32),
        scratch_types=[
            pltpu.VMEM((b_per_w,), jnp.int32),
            pltpu.VMEM((b_per_w, d), jnp.float32),
            pltpu.SemaphoreType.DMA,
        ],
    )
    def gk(idx_hbm, mem_hbm, out_hbm, idx_v, rows_v, sem):
        wid = lax.axis_index("s") * info.num_cores + lax.axis_index("c")
        base = wid * b_per_w
        pltpu.sync_copy(idx_hbm.at[pl.ds(base, b_per_w)], idx_v)
        pltpu.async_copy(mem_hbm.at[idx_v], rows_v, sem).wait()
        pltpu.sync_copy(rows_v, out_hbm.at[pl.ds(base, b_per_w)])

    return gk(idx_flat, memory)


def kernel(h, memory, activations, Wq, bq, Wk, bk, Wv, bv, Wo, bo, ln_g, ln_b,
           Wf, bf, activation_weight, gate_logit):
    B, T, d = h.shape
    N = memory.shape[0]
    K = min(FOCUS_K, N)
    H = N_HEADS
    Dh = d // H

    g2 = ln_g.reshape(1, d)
    b2 = ln_b.reshape(1, d)
    bq2 = bq.reshape(1, d)
    bf2 = bf.reshape(1, d)
    bk2 = bk.reshape(1, d)
    bv2 = bv.reshape(1, d)
    bo2 = bo.reshape(1, d)
    aw2 = activation_weight.reshape(1, 1)
    gl2 = gate_logit.reshape(1, 1)

    BTS = 512
    TS1 = T // BTS
    topk_idx = pl.pallas_call(
        functools.partial(_sumsel_body, bsz=B, n=N, k=K, t_steps=TS1,
                          inv_t=1.0 / T),
        grid=(B, TS1),
        in_specs=[
            pl.BlockSpec((1, BTS, d), lambda bb, tt: (bb, tt, 0)),
            pl.BlockSpec((1, d), lambda bb, tt: (0, 0)),
            pl.BlockSpec((1, d), lambda bb, tt: (0, 0)),
            pl.BlockSpec((d, d), lambda bb, tt: (0, 0)),
            pl.BlockSpec((1, d), lambda bb, tt: (0, 0)),
            pl.BlockSpec((B, N), lambda bb, tt: (0, 0)),
            pl.BlockSpec((1, 1), lambda bb, tt: (0, 0),
                         memory_space=pltpu.SMEM),
            pl.BlockSpec(memory_space=pl.ANY),
        ],
        out_specs=pl.BlockSpec((B, K), lambda bb, tt: (0, 0)),
        out_shape=jax.ShapeDtypeStruct((B, K), jnp.int32),
        scratch_shapes=[
            pltpu.VMEM((B, d), jnp.float32),
            pltpu.VMEM((N, d), jnp.float32),
            pltpu.SemaphoreType.DMA,
            pltpu.VMEM((B, N), jnp.float32),
        ],
    )(h, g2, b2, Wf, bf2, activations, aw2, memory)

    idx_flat = topk_idx.reshape(B * K)
    topk_mem = _sc_gather(memory, idx_flat, B * K, d)

    BT = 512
    TS = T // BT
    idx3 = topk_idx.reshape(B, 1, K)
    h_updated, full_attn = pl.pallas_call(
        functools.partial(_attn_body, heads=H, dh=Dh, k=K, n=N, t_total=T,
                          t_steps=TS),
        grid=(B, TS),
        in_specs=[
            pl.BlockSpec((1, BT, d), lambda bb, tt: (bb, tt, 0)),
            pl.BlockSpec((1, d), lambda bb, tt: (0, 0)),
            pl.BlockSpec((1, d), lambda bb, tt: (0, 0)),
            pl.BlockSpec((d, d), lambda bb, tt: (0, 0)),
            pl.BlockSpec((1, d), lambda bb, tt: (0, 0)),
            pl.BlockSpec((B * K, d), lambda bb, tt: (0, 0)),
            pl.BlockSpec((d, d), lambda bb, tt: (0, 0)),
            pl.BlockSpec((1, d), lambda bb, tt: (0, 0)),
            pl.BlockSpec((d, d), lambda bb, tt: (0, 0)),
            pl.BlockSpec((1, d), lambda bb, tt: (0, 0)),
            pl.BlockSpec((d, d), lambda bb, tt: (0, 0)),
            pl.BlockSpec((1, d), lambda bb, tt: (0, 0)),
            pl.BlockSpec((1, 1), lambda bb, tt: (0, 0), memory_space=pltpu.SMEM),
            pl.BlockSpec((1, 1, K), lambda bb, tt: (bb, 0, 0)),
        ],
        out_specs=[
            pl.BlockSpec((1, BT, d), lambda bb, tt: (bb, tt, 0)),
            pl.BlockSpec((1, 1, N), lambda bb, tt: (bb, 0, 0)),
        ],
        out_shape=[
            jax.ShapeDtypeStruct((B, T, d), jnp.float32),
            jax.ShapeDtypeStruct((B, 1, N), jnp.float32),
        ],
        scratch_shapes=[
            pltpu.VMEM((B * K, d), jnp.bfloat16),
            pltpu.VMEM((B * K, d), jnp.bfloat16),
            pltpu.VMEM((1, K), jnp.float32),
            pltpu.VMEM((d, d), jnp.bfloat16),
            pltpu.VMEM((d, d), jnp.bfloat16),
        ],
    )(h, g2, b2, Wq, bq2, topk_mem, Wk, bk2, Wv, bv2,
      Wo, bo2, gl2, idx3)
    full_attn = full_attn.reshape(B, N)

    return h_updated, full_attn


# speedup vs baseline: 2.0427x; 1.1370x over previous
import functools
import math

import jax
import jax.numpy as jnp
from jax import lax
from jax.experimental import pallas as pl
from jax.experimental.pallas import tpu as pltpu
from jax.experimental.pallas import tpu_sc as plsc

EPS = 1e-5
N_HEADS = 16
FOCUS_K = 64


def _ln(x, g, b):
    mu = jnp.mean(x, axis=-1, keepdims=True)
    var = jnp.mean((x - mu) ** 2, axis=-1, keepdims=True)
    return (x - mu) * lax.rsqrt(var + EPS) * g + b


def _sumsel_body(h_ref, g_ref, b_ref, wf_ref, bf_ref, act_ref, aw_ref,
                 mem_hbm, idx_ref, sums_ref, mem_v, sem, sel_ref,
                 *, bsz, n, k, t_steps, inv_t):
    bb = pl.program_id(0)
    t = pl.program_id(1)

    @pl.when((bb == 0) & (t == 0))
    def _():
        sums_ref[...] = jnp.zeros_like(sums_ref)
        pltpu.make_async_copy(mem_hbm, mem_v, sem).start()

    x = h_ref[0]
    xn = _ln(x, g_ref[...], b_ref[...])
    part = jnp.sum(xn, axis=0, keepdims=True) * inv_t
    biota = lax.broadcasted_iota(jnp.int32, (bsz, 1), 0)
    sums_ref[...] += jnp.where(biota == bb, part, 0.0)

    @pl.when((bb == bsz - 1) & (t == t_steps - 1))
    def _():
        pltpu.make_async_copy(mem_hbm, mem_v, sem).wait()
        fq = lax.dot_general(
            sums_ref[...], wf_ref[...], (((1,), (1,)), ((), ())),
            preferred_element_type=jnp.float32) + bf_ref[...]
        rel = lax.dot_general(fq, mem_v[...], (((1,), (1,)), ((), ())),
                              preferred_element_type=jnp.float32)
        sel_ref[...] = rel + aw_ref[0, 0] * act_ref[...]

        iota = lax.broadcasted_iota(jnp.int32, (bsz, n), 1)
        kcol = lax.broadcasted_iota(jnp.int32, (bsz, k), 1)

        def step(j, acc):
            vals = sel_ref[...]
            m = jnp.max(vals, axis=1, keepdims=True)
            idx = jnp.min(jnp.where(vals >= m, iota, n), axis=1, keepdims=True)
            sel_ref[...] = jnp.where(iota == idx, -jnp.inf, vals)
            return jnp.where(kcol == j, idx, acc)

        idx_ref[...] = lax.fori_loop(0, k, step,
                                     jnp.zeros((bsz, k), jnp.int32))


def _attn_body(h_ref, g_ref, b_ref, wq_ref, bq_ref, tm_ref, wk_ref, bk_ref,
               wv_ref, bv_ref, wo_ref, bo_ref, gate_ref, idx_ref,
               out_ref, fa_ref, kv_k, kv_v, asum_ref, wq_b, wo_b,
               *, heads, dh, k, n, t_total, t_steps):
    b = pl.program_id(0)
    t = pl.program_id(1)

    @pl.when((b == 0) & (t == 0))
    def _():
        tm = tm_ref[...]
        kf = lax.dot_general(tm, wk_ref[...], (((1,), (1,)), ((), ())),
                             preferred_element_type=jnp.float32) + bk_ref[...]
        vf = lax.dot_general(tm, wv_ref[...], (((1,), (1,)), ((), ())),
                             preferred_element_type=jnp.float32) + bv_ref[...]
        kv_k[...] = kf.astype(jnp.bfloat16)
        kv_v[...] = vf.astype(jnp.bfloat16)
        scale = 1.0 / math.sqrt(dh)
        wq_b[...] = (wq_ref[...] * scale).astype(jnp.bfloat16)
        wo_b[...] = wo_ref[...].astype(jnp.bfloat16)

    x = h_ref[0]
    xn = _ln(x, g_ref[...], b_ref[...])
    scale = 1.0 / math.sqrt(dh)
    q = lax.dot_general(xn.astype(jnp.bfloat16), wq_b[...],
                        (((1,), (1,)), ((), ())),
                        preferred_element_type=jnp.float32) \
        + bq_ref[...] * scale
    qb = q.astype(jnp.bfloat16)
    kk = kv_k[pl.ds(pl.multiple_of(b * k, k), k), :]
    vv = kv_v[pl.ds(pl.multiple_of(b * k, k), k), :]
    d_model = heads * dh
    s_parts = []
    for hh in range(heads):
        qh = qb[:, hh * dh:(hh + 1) * dh]
        kh = kk[:, hh * dh:(hh + 1) * dh]
        s_parts.append(lax.dot_general(qh, kh, (((1,), (1,)), ((), ())),
                                       preferred_element_type=jnp.float32))
    s_all = jnp.concatenate(s_parts, axis=1)
    e_all = jnp.exp(s_all)
    eb = e_all.astype(jnp.bfloat16)
    seg_r = lax.broadcasted_iota(jnp.int32, (heads * k, heads), 0) // k
    seg_c = lax.broadcasted_iota(jnp.int32, (heads * k, heads), 1)
    seg = (seg_r == seg_c).astype(jnp.bfloat16)
    rs = lax.dot_general(eb, seg, (((1,), (0,)), ((), ())),
                         preferred_element_type=jnp.float32)
    r = 1.0 / rs
    ex_r = lax.broadcasted_iota(jnp.int32, (heads, heads * k), 0)
    ex_c = lax.broadcasted_iota(jnp.int32, (heads, heads * k), 1) // k
    exf = (ex_r == ex_c).astype(jnp.float32)
    rexp = lax.dot_general(r, exf, (((1,), (0,)), ((), ())),
                           preferred_element_type=jnp.float32)
    p_all = e_all * rexp
    pb = p_all.astype(jnp.bfloat16)
    o_parts = []
    for hh in range(heads):
        ph = pb[:, hh * k:(hh + 1) * k]
        vh = vv[:, hh * dh:(hh + 1) * dh]
        o_parts.append(lax.dot_general(ph, vh, (((1,), (0,)), ((), ())),
                                       preferred_element_type=jnp.float32))
    att = jnp.concatenate(o_parts, axis=1).astype(jnp.bfloat16)
    o = lax.dot_general(att, wo_b[...], (((1,), (1,)), ((), ())),
                        preferred_element_type=jnp.float32) + bo_ref[...]
    gate = 1.0 / (1.0 + jnp.exp(-gate_ref[0, 0]))
    out_ref[0] = x + gate * o

    @pl.when(t == 0)
    def _():
        asum_ref[...] = jnp.zeros_like(asum_ref)

    ones_row = jnp.ones((1, p_all.shape[0]), jnp.float32)
    asum_ref[...] += lax.dot_general(
        ones_row, p_all, (((1,), (0,)), ((), ())),
        preferred_element_type=jnp.float32) * (1.0 / (heads * t_total))

    @pl.when(t == t_steps - 1)
    def _():
        idx = idx_ref[0]
        f_r = lax.broadcasted_iota(jnp.int32, (heads * k, k), 0)
        f_c = lax.broadcasted_iota(jnp.int32, (heads * k, k), 1)
        fold = (f_r % k == f_c).astype(jnp.float32)
        vals = lax.dot_general(asum_ref[...], fold, (((1,), (0,)), ((), ())),
                               preferred_element_type=jnp.float32)
        iota = lax.broadcasted_iota(jnp.int32, (k, n), 1)
        onehot = (iota == idx.reshape(k, 1)).astype(jnp.float32)
        fa_ref[0] = lax.dot_general(vals, onehot, (((1,), (0,)), ((), ())),
                                    preferred_element_type=jnp.float32)


def _sc_gather(memory, idx_flat, rows, d):
    info = plsc.get_sparse_core_info()
    nw = info.num_cores * info.num_subcores
    b_per_w = rows // nw
    mesh = plsc.VectorSubcoreMesh(core_axis_name="c", subcore_axis_name="s")

    @functools.partial(
        pl.kernel, mesh=mesh,
        out_type=jax.ShapeDtypeStruct((rows, d), jnp.float32),
        scratch_types=[
            pltpu.VMEM((b_per_w,), jnp.int32),
            pltpu.VMEM((b_per_w, d), jnp.float32),
            pltpu.SemaphoreType.DMA,
        ],
    )
    def gk(idx_hbm, mem_hbm, out_hbm, idx_v, rows_v, sem):
        wid = lax.axis_index("s") * info.num_cores + lax.axis_index("c")
        base = wid * b_per_w
        pltpu.sync_copy(idx_hbm.at[pl.ds(base, b_per_w)], idx_v)
        pltpu.async_copy(mem_hbm.at[idx_v], rows_v, sem).wait()
        pltpu.sync_copy(rows_v, out_hbm.at[pl.ds(base, b_per_w)])

    return gk(idx_flat, memory)


def kernel(h, memory, activations, Wq, bq, Wk, bk, Wv, bv, Wo, bo, ln_g, ln_b,
           Wf, bf, activation_weight, gate_logit):
    B, T, d = h.shape
    N = memory.shape[0]
    K = min(FOCUS_K, N)
    H = N_HEADS
    Dh = d // H

    g2 = ln_g.reshape(1, d)
    b2 = ln_b.reshape(1, d)
    bq2 = bq.reshape(1, d)
    bf2 = bf.reshape(1, d)
    bk2 = bk.reshape(1, d)
    bv2 = bv.reshape(1, d)
    bo2 = bo.reshape(1, d)
    aw2 = activation_weight.reshape(1, 1)
    gl2 = gate_logit.reshape(1, 1)

    BTS = 512
    TS1 = T // BTS
    topk_idx = pl.pallas_call(
        functools.partial(_sumsel_body, bsz=B, n=N, k=K, t_steps=TS1,
                          inv_t=1.0 / T),
        grid=(B, TS1),
        in_specs=[
            pl.BlockSpec((1, BTS, d), lambda bb, tt: (bb, tt, 0)),
            pl.BlockSpec((1, d), lambda bb, tt: (0, 0)),
            pl.BlockSpec((1, d), lambda bb, tt: (0, 0)),
            pl.BlockSpec((d, d), lambda bb, tt: (0, 0)),
            pl.BlockSpec((1, d), lambda bb, tt: (0, 0)),
            pl.BlockSpec((B, N), lambda bb, tt: (0, 0)),
            pl.BlockSpec((1, 1), lambda bb, tt: (0, 0),
                         memory_space=pltpu.SMEM),
            pl.BlockSpec(memory_space=pl.ANY),
        ],
        out_specs=pl.BlockSpec((B, K), lambda bb, tt: (0, 0)),
        out_shape=jax.ShapeDtypeStruct((B, K), jnp.int32),
        scratch_shapes=[
            pltpu.VMEM((B, d), jnp.float32),
            pltpu.VMEM((N, d), jnp.float32),
            pltpu.SemaphoreType.DMA,
            pltpu.VMEM((B, N), jnp.float32),
        ],
    )(h, g2, b2, Wf, bf2, activations, aw2, memory)

    idx_flat = topk_idx.reshape(B * K)
    topk_mem = _sc_gather(memory, idx_flat, B * K, d)

    BT = 512
    TS = T // BT
    idx3 = topk_idx.reshape(B, 1, K)
    h_updated, full_attn = pl.pallas_call(
        functools.partial(_attn_body, heads=H, dh=Dh, k=K, n=N, t_total=T,
                          t_steps=TS),
        grid=(B, TS),
        in_specs=[
            pl.BlockSpec((1, BT, d), lambda bb, tt: (bb, tt, 0)),
            pl.BlockSpec((1, d), lambda bb, tt: (0, 0)),
            pl.BlockSpec((1, d), lambda bb, tt: (0, 0)),
            pl.BlockSpec((d, d), lambda bb, tt: (0, 0)),
            pl.BlockSpec((1, d), lambda bb, tt: (0, 0)),
            pl.BlockSpec((B * K, d), lambda bb, tt: (0, 0)),
            pl.BlockSpec((d, d), lambda bb, tt: (0, 0)),
            pl.BlockSpec((1, d), lambda bb, tt: (0, 0)),
            pl.BlockSpec((d, d), lambda bb, tt: (0, 0)),
            pl.BlockSpec((1, d), lambda bb, tt: (0, 0)),
            pl.BlockSpec((d, d), lambda bb, tt: (0, 0)),
            pl.BlockSpec((1, d), lambda bb, tt: (0, 0)),
            pl.BlockSpec((1, 1), lambda bb, tt: (0, 0), memory_space=pltpu.SMEM),
            pl.BlockSpec((1, 1, K), lambda bb, tt: (bb, 0, 0)),
        ],
        out_specs=[
            pl.BlockSpec((1, BT, d), lambda bb, tt: (bb, tt, 0)),
            pl.BlockSpec((1, 1, N), lambda bb, tt: (bb, 0, 0)),
        ],
        out_shape=[
            jax.ShapeDtypeStruct((B, T, d), jnp.float32),
            jax.ShapeDtypeStruct((B, 1, N), jnp.float32),
        ],
        scratch_shapes=[
            pltpu.VMEM((B * K, d), jnp.bfloat16),
            pltpu.VMEM((B * K, d), jnp.bfloat16),
            pltpu.VMEM((1, H * K), jnp.float32),
            pltpu.VMEM((d, d), jnp.bfloat16),
            pltpu.VMEM((d, d), jnp.bfloat16),
        ],
    )(h, g2, b2, Wq, bq2, topk_mem, Wk, bk2, Wv, bv2,
      Wo, bo2, gl2, idx3)
    full_attn = full_attn.reshape(B, N)

    return h_updated, full_attn


# speedup vs baseline: 2.1359x; 1.0456x over previous
import functools
import math

import jax
import jax.numpy as jnp
from jax import lax
from jax.experimental import pallas as pl
from jax.experimental.pallas import tpu as pltpu
from jax.experimental.pallas import tpu_sc as plsc

EPS = 1e-5
N_HEADS = 16
FOCUS_K = 64


def _ln(x, g, b):
    mu = jnp.mean(x, axis=-1, keepdims=True)
    var = jnp.mean((x - mu) ** 2, axis=-1, keepdims=True)
    return (x - mu) * lax.rsqrt(var + EPS) * g + b


def _sumsel_body(h_ref, g_ref, b_ref, wf_ref, bf_ref, act_ref, aw_ref,
                 mem_hbm, idx_ref, sums_ref, mem_v, sem, sel_ref,
                 *, bsz, n, k, t_steps, inv_t):
    bb = pl.program_id(0)
    t = pl.program_id(1)

    @pl.when((bb == 0) & (t == 0))
    def _():
        sums_ref[...] = jnp.zeros_like(sums_ref)
        pltpu.make_async_copy(mem_hbm, mem_v, sem).start()

    x = h_ref[0]
    xn = _ln(x, g_ref[...], b_ref[...])
    part = jnp.sum(xn, axis=0, keepdims=True) * inv_t
    biota = lax.broadcasted_iota(jnp.int32, (bsz, 1), 0)
    sums_ref[...] += jnp.where(biota == bb, part, 0.0)

    @pl.when((bb == bsz - 1) & (t == t_steps - 1))
    def _():
        pltpu.make_async_copy(mem_hbm, mem_v, sem).wait()
        fq = lax.dot_general(
            sums_ref[...], wf_ref[...], (((1,), (1,)), ((), ())),
            preferred_element_type=jnp.float32) + bf_ref[...]
        rel = lax.dot_general(fq, mem_v[...], (((1,), (1,)), ((), ())),
                              preferred_element_type=jnp.float32)
        sel_ref[...] = rel + aw_ref[0, 0] * act_ref[...]

        iota = lax.broadcasted_iota(jnp.int32, (bsz, n), 1)
        kcol = lax.broadcasted_iota(jnp.int32, (bsz, k), 1)

        def step(j, acc):
            vals = sel_ref[...]
            m = jnp.max(vals, axis=1, keepdims=True)
            idx = jnp.min(jnp.where(vals >= m, iota, n), axis=1, keepdims=True)
            sel_ref[...] = jnp.where(iota == idx, -jnp.inf, vals)
            return jnp.where(kcol == j, idx, acc)

        idx_ref[...] = lax.fori_loop(0, k, step,
                                     jnp.zeros((bsz, k), jnp.int32))


def _attn_body(h_ref, g_ref, b_ref, wq_ref, bq_ref, tm_ref, wk_ref, bk_ref,
               wv_ref, bv_ref, wo_ref, bo_ref, gate_ref, idx_ref,
               out_ref, fa_ref, kv_k, kv_v, asum_ref, wq_b, wo_b,
               *, heads, dh, k, n, t_total, t_steps):
    b = pl.program_id(0)
    t = pl.program_id(1)

    @pl.when((b == 0) & (t == 0))
    def _():
        tm = tm_ref[...]
        kf = lax.dot_general(tm, wk_ref[...], (((1,), (1,)), ((), ())),
                             preferred_element_type=jnp.float32) + bk_ref[...]
        vf = lax.dot_general(tm, wv_ref[...], (((1,), (1,)), ((), ())),
                             preferred_element_type=jnp.float32) + bv_ref[...]
        kv_k[...] = kf.astype(jnp.bfloat16)
        kv_v[...] = vf.astype(jnp.bfloat16)
        scale = 1.0 / math.sqrt(dh)
        wq_b[...] = (wq_ref[...] * scale).astype(jnp.bfloat16)
        wo_b[...] = wo_ref[...].astype(jnp.bfloat16)

    x = h_ref[0]
    xn = _ln(x, g_ref[...], b_ref[...])
    scale = 1.0 / math.sqrt(dh)
    q = lax.dot_general(xn.astype(jnp.bfloat16), wq_b[...],
                        (((1,), (1,)), ((), ())),
                        preferred_element_type=jnp.float32) \
        + bq_ref[...] * scale
    qb = q.astype(jnp.bfloat16)
    kk = kv_k[pl.ds(pl.multiple_of(b * k, k), k), :]
    vv = kv_v[pl.ds(pl.multiple_of(b * k, k), k), :]
    d_model = heads * dh
    s_parts = []
    for hh in range(heads):
        qh = qb[:, hh * dh:(hh + 1) * dh]
        kh = kk[:, hh * dh:(hh + 1) * dh]
        s_parts.append(lax.dot_general(qh, kh, (((1,), (1,)), ((), ())),
                                       preferred_element_type=jnp.float32))
    s_all = jnp.concatenate(s_parts, axis=1)
    e_all = jnp.exp(s_all)
    eb = e_all.astype(jnp.bfloat16)
    seg_r = lax.broadcasted_iota(jnp.int32, (heads * k, heads), 0) // k
    seg_c = lax.broadcasted_iota(jnp.int32, (heads * k, heads), 1)
    seg = (seg_r == seg_c).astype(jnp.bfloat16)
    rs = lax.dot_general(eb, seg, (((1,), (0,)), ((), ())),
                         preferred_element_type=jnp.float32)
    r = 1.0 / rs
    ex_r = lax.broadcasted_iota(jnp.int32, (heads, heads * k), 0)
    ex_c = lax.broadcasted_iota(jnp.int32, (heads, heads * k), 1) // k
    exf = (ex_r == ex_c).astype(jnp.float32)
    rexp = lax.dot_general(r, exf, (((1,), (0,)), ((), ())),
                           preferred_element_type=jnp.float32)
    p_all = e_all * rexp
    pb = p_all.astype(jnp.bfloat16)
    o_parts = []
    for hh in range(heads):
        ph = pb[:, hh * k:(hh + 1) * k]
        vh = vv[:, hh * dh:(hh + 1) * dh]
        o_parts.append(lax.dot_general(ph, vh, (((1,), (0,)), ((), ())),
                                       preferred_element_type=jnp.float32))
    att = jnp.concatenate(o_parts, axis=1).astype(jnp.bfloat16)
    o = lax.dot_general(att, wo_b[...], (((1,), (1,)), ((), ())),
                        preferred_element_type=jnp.float32) + bo_ref[...]
    gate = 1.0 / (1.0 + jnp.exp(-gate_ref[0, 0]))
    out_ref[0] = x + gate * o

    @pl.when(t == 0)
    def _():
        asum_ref[...] = jnp.zeros_like(asum_ref)

    ones_row = jnp.ones((1, p_all.shape[0]), jnp.float32)
    asum_ref[...] += lax.dot_general(
        ones_row, p_all, (((1,), (0,)), ((), ())),
        preferred_element_type=jnp.float32) * (1.0 / (heads * t_total))

    @pl.when(t == t_steps - 1)
    def _():
        idx = idx_ref[0]
        f_r = lax.broadcasted_iota(jnp.int32, (heads * k, k), 0)
        f_c = lax.broadcasted_iota(jnp.int32, (heads * k, k), 1)
        fold = (f_r % k == f_c).astype(jnp.float32)
        vals = lax.dot_general(asum_ref[...], fold, (((1,), (0,)), ((), ())),
                               preferred_element_type=jnp.float32)
        iota = lax.broadcasted_iota(jnp.int32, (k, n), 1)
        onehot = (iota == idx.reshape(k, 1)).astype(jnp.float32)
        fa_ref[0] = lax.dot_general(vals, onehot, (((1,), (0,)), ((), ())),
                                    preferred_element_type=jnp.float32)


def _sc_gather(memory, idx_flat, rows, d):
    info = plsc.get_sparse_core_info()
    nw = info.num_cores * info.num_subcores
    b_per_w = rows // nw
    mesh = plsc.VectorSubcoreMesh(core_axis_name="c", subcore_axis_name="s")

    @functools.partial(
        pl.kernel, mesh=mesh,
        out_type=jax.ShapeDtypeStruct((rows, d), jnp.float32),
        scratch_types=[
            pltpu.VMEM((b_per_w,), jnp.int32),
            pltpu.VMEM((b_per_w, d), jnp.float32),
            pltpu.SemaphoreType.DMA,
        ],
    )
    def gk(idx_hbm, mem_hbm, out_hbm, idx_v, rows_v, sem):
        wid = lax.axis_index("s") * info.num_cores + lax.axis_index("c")
        base = wid * b_per_w
        pltpu.sync_copy(idx_hbm.at[pl.ds(base, b_per_w)], idx_v)
        pltpu.async_copy(mem_hbm.at[idx_v], rows_v, sem).wait()
        pltpu.sync_copy(rows_v, out_hbm.at[pl.ds(base, b_per_w)])

    return gk(idx_flat, memory)


def kernel(h, memory, activations, Wq, bq, Wk, bk, Wv, bv, Wo, bo, ln_g, ln_b,
           Wf, bf, activation_weight, gate_logit):
    B, T, d = h.shape
    N = memory.shape[0]
    K = min(FOCUS_K, N)
    H = N_HEADS
    Dh = d // H

    g2 = ln_g.reshape(1, d)
    b2 = ln_b.reshape(1, d)
    bq2 = bq.reshape(1, d)
    bf2 = bf.reshape(1, d)
    bk2 = bk.reshape(1, d)
    bv2 = bv.reshape(1, d)
    bo2 = bo.reshape(1, d)
    aw2 = activation_weight.reshape(1, 1)
    gl2 = gate_logit.reshape(1, 1)

    BTS = 1024
    TS1 = T // BTS
    topk_idx = pl.pallas_call(
        functools.partial(_sumsel_body, bsz=B, n=N, k=K, t_steps=TS1,
                          inv_t=1.0 / T),
        grid=(B, TS1),
        in_specs=[
            pl.BlockSpec((1, BTS, d), lambda bb, tt: (bb, tt, 0)),
            pl.BlockSpec((1, d), lambda bb, tt: (0, 0)),
            pl.BlockSpec((1, d), lambda bb, tt: (0, 0)),
            pl.BlockSpec((d, d), lambda bb, tt: (0, 0)),
            pl.BlockSpec((1, d), lambda bb, tt: (0, 0)),
            pl.BlockSpec((B, N), lambda bb, tt: (0, 0)),
            pl.BlockSpec((1, 1), lambda bb, tt: (0, 0),
                         memory_space=pltpu.SMEM),
            pl.BlockSpec(memory_space=pl.ANY),
        ],
        out_specs=pl.BlockSpec((B, K), lambda bb, tt: (0, 0)),
        out_shape=jax.ShapeDtypeStruct((B, K), jnp.int32),
        scratch_shapes=[
            pltpu.VMEM((B, d), jnp.float32),
            pltpu.VMEM((N, d), jnp.float32),
            pltpu.SemaphoreType.DMA,
            pltpu.VMEM((B, N), jnp.float32),
        ],
    )(h, g2, b2, Wf, bf2, activations, aw2, memory)

    idx_flat = topk_idx.reshape(B * K)
    topk_mem = _sc_gather(memory, idx_flat, B * K, d)

    BT = 1024
    TS = T // BT
    idx3 = topk_idx.reshape(B, 1, K)
    h_updated, full_attn = pl.pallas_call(
        functools.partial(_attn_body, heads=H, dh=Dh, k=K, n=N, t_total=T,
                          t_steps=TS),
        grid=(B, TS),
        in_specs=[
            pl.BlockSpec((1, BT, d), lambda bb, tt: (bb, tt, 0)),
            pl.BlockSpec((1, d), lambda bb, tt: (0, 0)),
            pl.BlockSpec((1, d), lambda bb, tt: (0, 0)),
            pl.BlockSpec((d, d), lambda bb, tt: (0, 0)),
            pl.BlockSpec((1, d), lambda bb, tt: (0, 0)),
            pl.BlockSpec((B * K, d), lambda bb, tt: (0, 0)),
            pl.BlockSpec((d, d), lambda bb, tt: (0, 0)),
            pl.BlockSpec((1, d), lambda bb, tt: (0, 0)),
            pl.BlockSpec((d, d), lambda bb, tt: (0, 0)),
            pl.BlockSpec((1, d), lambda bb, tt: (0, 0)),
            pl.BlockSpec((d, d), lambda bb, tt: (0, 0)),
            pl.BlockSpec((1, d), lambda bb, tt: (0, 0)),
            pl.BlockSpec((1, 1), lambda bb, tt: (0, 0), memory_space=pltpu.SMEM),
            pl.BlockSpec((1, 1, K), lambda bb, tt: (bb, 0, 0)),
        ],
        out_specs=[
            pl.BlockSpec((1, BT, d), lambda bb, tt: (bb, tt, 0)),
            pl.BlockSpec((1, 1, N), lambda bb, tt: (bb, 0, 0)),
        ],
        out_shape=[
            jax.ShapeDtypeStruct((B, T, d), jnp.float32),
            jax.ShapeDtypeStruct((B, 1, N), jnp.float32),
        ],
        scratch_shapes=[
            pltpu.VMEM((B * K, d), jnp.bfloat16),
            pltpu.VMEM((B * K, d), jnp.bfloat16),
            pltpu.VMEM((1, H * K), jnp.float32),
            pltpu.VMEM((d, d), jnp.bfloat16),
            pltpu.VMEM((d, d), jnp.bfloat16),
        ],
    )(h, g2, b2, Wq, bq2, topk_mem, Wk, bk2, Wv, bv2,
      Wo, bo2, gl2, idx3)
    full_attn = full_attn.reshape(B, N)

    return h_updated, full_attn


# speedup vs baseline: 2.1440x; 1.0038x over previous
import functools
import math

import jax
import jax.numpy as jnp
from jax import lax
from jax.experimental import pallas as pl
from jax.experimental.pallas import tpu as pltpu
from jax.experimental.pallas import tpu_sc as plsc

EPS = 1e-5
N_HEADS = 16
FOCUS_K = 64


def _ln(x, g, b):
    mu = jnp.mean(x, axis=-1, keepdims=True)
    var = jnp.mean((x - mu) ** 2, axis=-1, keepdims=True)
    return (x - mu) * lax.rsqrt(var + EPS) * g + b


def _sumsel_body(h_ref, g_ref, b_ref, wf_ref, bf_ref, act_ref, aw_ref,
                 mem_hbm, idx_ref, sums_ref, mem_v, sem, sel_ref,
                 *, bsz, n, k, t_steps, inv_t):
    bb = pl.program_id(0)
    t = pl.program_id(1)

    @pl.when((bb == 0) & (t == 0))
    def _():
        sums_ref[...] = jnp.zeros_like(sums_ref)
        pltpu.make_async_copy(mem_hbm, mem_v, sem).start()

    x = h_ref[0]
    xn = _ln(x, g_ref[...], b_ref[...])
    part = jnp.sum(xn, axis=0, keepdims=True) * inv_t
    biota = lax.broadcasted_iota(jnp.int32, (bsz, 1), 0)
    sums_ref[...] += jnp.where(biota == bb, part, 0.0)

    @pl.when((bb == bsz - 1) & (t == t_steps - 1))
    def _():
        pltpu.make_async_copy(mem_hbm, mem_v, sem).wait()
        fq = lax.dot_general(
            sums_ref[...], wf_ref[...], (((1,), (1,)), ((), ())),
            preferred_element_type=jnp.float32) + bf_ref[...]
        rel = lax.dot_general(fq, mem_v[...], (((1,), (1,)), ((), ())),
                              preferred_element_type=jnp.float32)
        sel_ref[...] = rel + aw_ref[0, 0] * act_ref[...]

        iota = lax.broadcasted_iota(jnp.int32, (bsz, n), 1)
        kcol = lax.broadcasted_iota(jnp.int32, (bsz, k), 1)

        def step(j, acc):
            vals = sel_ref[...]
            m = jnp.max(vals, axis=1, keepdims=True)
            idx = jnp.min(jnp.where(vals >= m, iota, n), axis=1, keepdims=True)
            sel_ref[...] = jnp.where(iota == idx, -jnp.inf, vals)
            return jnp.where(kcol == j, idx, acc)

        idx_ref[...] = lax.fori_loop(0, k, step,
                                     jnp.zeros((bsz, k), jnp.int32))


def _attn_body(h_ref, g_ref, b_ref, wq_ref, bq_ref, tm_ref, wk_ref, bk_ref,
               wv_ref, bv_ref, wo_ref, bo_ref, gate_ref, idx_ref,
               out_ref, fa_ref, kv_k, kv_v, asum_ref, wq_b, wo_b,
               *, heads, dh, k, n, t_total, t_steps):
    b = pl.program_id(0)
    t = pl.program_id(1)

    @pl.when((b == 0) & (t == 0))
    def _():
        tm = tm_ref[...]
        kf = lax.dot_general(tm, wk_ref[...], (((1,), (1,)), ((), ())),
                             preferred_element_type=jnp.float32) + bk_ref[...]
        vf = lax.dot_general(tm, wv_ref[...], (((1,), (1,)), ((), ())),
                             preferred_element_type=jnp.float32) + bv_ref[...]
        kv_k[...] = kf.astype(jnp.bfloat16)
        kv_v[...] = vf.astype(jnp.bfloat16)
        scale = 1.0 / math.sqrt(dh)
        wq_b[...] = (wq_ref[...] * scale).astype(jnp.bfloat16)
        wo_b[...] = wo_ref[...].astype(jnp.bfloat16)

    kk = kv_k[pl.ds(pl.multiple_of(b * k, k), k), :]
    vv = kv_v[pl.ds(pl.multiple_of(b * k, k), k), :]
    scale = 1.0 / math.sqrt(dh)
    gate = 1.0 / (1.0 + jnp.exp(-gate_ref[0, 0]))
    seg_r = lax.broadcasted_iota(jnp.int32, (heads * k, heads), 0) // k
    seg_c = lax.broadcasted_iota(jnp.int32, (heads * k, heads), 1)
    seg = (seg_r == seg_c).astype(jnp.bfloat16)
    ex_r = lax.broadcasted_iota(jnp.int32, (heads, heads * k), 0)
    ex_c = lax.broadcasted_iota(jnp.int32, (heads, heads * k), 1) // k
    exf = (ex_r == ex_c).astype(jnp.float32)

    @pl.when(t == 0)
    def _():
        asum_ref[...] = jnp.zeros_like(asum_ref)

    bt = out_ref.shape[1]
    nhalf = 2
    hrows = bt // nhalf
    for half in range(nhalf):
        rsl = slice(half * hrows, (half + 1) * hrows)
        x = h_ref[0, rsl, :]
        xn = _ln(x, g_ref[...], b_ref[...])
        q = lax.dot_general(xn.astype(jnp.bfloat16), wq_b[...],
                            (((1,), (1,)), ((), ())),
                            preferred_element_type=jnp.float32) \
            + bq_ref[...] * scale
        qb = q.astype(jnp.bfloat16)
        s_parts = []
        for hh in range(heads):
            qh = qb[:, hh * dh:(hh + 1) * dh]
            kh = kk[:, hh * dh:(hh + 1) * dh]
            s_parts.append(lax.dot_general(qh, kh, (((1,), (1,)), ((), ())),
                                           preferred_element_type=jnp.float32))
        s_all = jnp.concatenate(s_parts, axis=1)
        e_all = jnp.exp(s_all)
        eb = e_all.astype(jnp.bfloat16)
        rs = lax.dot_general(eb, seg, (((1,), (0,)), ((), ())),
                             preferred_element_type=jnp.float32)
        r = 1.0 / rs
        rexp = lax.dot_general(r, exf, (((1,), (0,)), ((), ())),
                               preferred_element_type=jnp.float32)
        p_all = e_all * rexp
        pb = p_all.astype(jnp.bfloat16)
        o_parts = []
        for hh in range(heads):
            ph = pb[:, hh * k:(hh + 1) * k]
            vh = vv[:, hh * dh:(hh + 1) * dh]
            o_parts.append(lax.dot_general(ph, vh, (((1,), (0,)), ((), ())),
                                           preferred_element_type=jnp.float32))
        att = jnp.concatenate(o_parts, axis=1).astype(jnp.bfloat16)
        o = lax.dot_general(att, wo_b[...], (((1,), (1,)), ((), ())),
                            preferred_element_type=jnp.float32) + bo_ref[...]
        out_ref[0, rsl, :] = x + gate * o
        ones_row = jnp.ones((1, hrows), jnp.float32)
        asum_ref[...] += lax.dot_general(
            ones_row, p_all, (((1,), (0,)), ((), ())),
            preferred_element_type=jnp.float32) * (1.0 / (heads * t_total))

    @pl.when(t == t_steps - 1)
    def _():
        idx = idx_ref[0]
        f_r = lax.broadcasted_iota(jnp.int32, (heads * k, k), 0)
        f_c = lax.broadcasted_iota(jnp.int32, (heads * k, k), 1)
        fold = (f_r % k == f_c).astype(jnp.float32)
        vals = lax.dot_general(asum_ref[...], fold, (((1,), (0,)), ((), ())),
                               preferred_element_type=jnp.float32)
        iota = lax.broadcasted_iota(jnp.int32, (k, n), 1)
        onehot = (iota == idx.reshape(k, 1)).astype(jnp.float32)
        fa_ref[0] = lax.dot_general(vals, onehot, (((1,), (0,)), ((), ())),
                                    preferred_element_type=jnp.float32)


def _sc_gather(memory, idx_flat, rows, d):
    info = plsc.get_sparse_core_info()
    nw = info.num_cores * info.num_subcores
    b_per_w = rows // nw
    mesh = plsc.VectorSubcoreMesh(core_axis_name="c", subcore_axis_name="s")

    @functools.partial(
        pl.kernel, mesh=mesh,
        out_type=jax.ShapeDtypeStruct((rows, d), jnp.float32),
        scratch_types=[
            pltpu.VMEM((b_per_w,), jnp.int32),
            pltpu.VMEM((b_per_w, d), jnp.float32),
            pltpu.SemaphoreType.DMA,
        ],
    )
    def gk(idx_hbm, mem_hbm, out_hbm, idx_v, rows_v, sem):
        wid = lax.axis_index("s") * info.num_cores + lax.axis_index("c")
        base = wid * b_per_w
        pltpu.sync_copy(idx_hbm.at[pl.ds(base, b_per_w)], idx_v)
        pltpu.async_copy(mem_hbm.at[idx_v], rows_v, sem).wait()
        pltpu.sync_copy(rows_v, out_hbm.at[pl.ds(base, b_per_w)])

    return gk(idx_flat, memory)


def kernel(h, memory, activations, Wq, bq, Wk, bk, Wv, bv, Wo, bo, ln_g, ln_b,
           Wf, bf, activation_weight, gate_logit):
    B, T, d = h.shape
    N = memory.shape[0]
    K = min(FOCUS_K, N)
    H = N_HEADS
    Dh = d // H

    g2 = ln_g.reshape(1, d)
    b2 = ln_b.reshape(1, d)
    bq2 = bq.reshape(1, d)
    bf2 = bf.reshape(1, d)
    bk2 = bk.reshape(1, d)
    bv2 = bv.reshape(1, d)
    bo2 = bo.reshape(1, d)
    aw2 = activation_weight.reshape(1, 1)
    gl2 = gate_logit.reshape(1, 1)

    BTS = 1024
    TS1 = T // BTS
    topk_idx = pl.pallas_call(
        functools.partial(_sumsel_body, bsz=B, n=N, k=K, t_steps=TS1,
                          inv_t=1.0 / T),
        grid=(B, TS1),
        in_specs=[
            pl.BlockSpec((1, BTS, d), lambda bb, tt: (bb, tt, 0)),
            pl.BlockSpec((1, d), lambda bb, tt: (0, 0)),
            pl.BlockSpec((1, d), lambda bb, tt: (0, 0)),
            pl.BlockSpec((d, d), lambda bb, tt: (0, 0)),
            pl.BlockSpec((1, d), lambda bb, tt: (0, 0)),
            pl.BlockSpec((B, N), lambda bb, tt: (0, 0)),
            pl.BlockSpec((1, 1), lambda bb, tt: (0, 0),
                         memory_space=pltpu.SMEM),
            pl.BlockSpec(memory_space=pl.ANY),
        ],
        out_specs=pl.BlockSpec((B, K), lambda bb, tt: (0, 0)),
        out_shape=jax.ShapeDtypeStruct((B, K), jnp.int32),
        scratch_shapes=[
            pltpu.VMEM((B, d), jnp.float32),
            pltpu.VMEM((N, d), jnp.float32),
            pltpu.SemaphoreType.DMA,
            pltpu.VMEM((B, N), jnp.float32),
        ],
    )(h, g2, b2, Wf, bf2, activations, aw2, memory)

    idx_flat = topk_idx.reshape(B * K)
    topk_mem = _sc_gather(memory, idx_flat, B * K, d)

    BT = 1024
    TS = T // BT
    idx3 = topk_idx.reshape(B, 1, K)
    h_updated, full_attn = pl.pallas_call(
        functools.partial(_attn_body, heads=H, dh=Dh, k=K, n=N, t_total=T,
                          t_steps=TS),
        grid=(B, TS),
        in_specs=[
            pl.BlockSpec((1, BT, d), lambda bb, tt: (bb, tt, 0)),
            pl.BlockSpec((1, d), lambda bb, tt: (0, 0)),
            pl.BlockSpec((1, d), lambda bb, tt: (0, 0)),
            pl.BlockSpec((d, d), lambda bb, tt: (0, 0)),
            pl.BlockSpec((1, d), lambda bb, tt: (0, 0)),
            pl.BlockSpec((B * K, d), lambda bb, tt: (0, 0)),
            pl.BlockSpec((d, d), lambda bb, tt: (0, 0)),
            pl.BlockSpec((1, d), lambda bb, tt: (0, 0)),
            pl.BlockSpec((d, d), lambda bb, tt: (0, 0)),
            pl.BlockSpec((1, d), lambda bb, tt: (0, 0)),
            pl.BlockSpec((d, d), lambda bb, tt: (0, 0)),
            pl.BlockSpec((1, d), lambda bb, tt: (0, 0)),
            pl.BlockSpec((1, 1), lambda bb, tt: (0, 0), memory_space=pltpu.SMEM),
            pl.BlockSpec((1, 1, K), lambda bb, tt: (bb, 0, 0)),
        ],
        out_specs=[
            pl.BlockSpec((1, BT, d), lambda bb, tt: (bb, tt, 0)),
            pl.BlockSpec((1, 1, N), lambda bb, tt: (bb, 0, 0)),
        ],
        out_shape=[
            jax.ShapeDtypeStruct((B, T, d), jnp.float32),
            jax.ShapeDtypeStruct((B, 1, N), jnp.float32),
        ],
        scratch_shapes=[
            pltpu.VMEM((B * K, d), jnp.bfloat16),
            pltpu.VMEM((B * K, d), jnp.bfloat16),
            pltpu.VMEM((1, H * K), jnp.float32),
            pltpu.VMEM((d, d), jnp.bfloat16),
            pltpu.VMEM((d, d), jnp.bfloat16),
        ],
    )(h, g2, b2, Wq, bq2, topk_mem, Wk, bk2, Wv, bv2,
      Wo, bo2, gl2, idx3)
    full_attn = full_attn.reshape(B, N)

    return h_updated, full_attn
